# baseline (speedup 1.0000x reference)
import functools

import jax
import jax.numpy as jnp
from jax import lax
from jax.experimental import pallas as pl
from jax.experimental.pallas import tpu as pltpu
from jax.experimental.pallas import tpu_sc as plsc

N_EXP = 8
TOPK = 2
NC = 2
NS = 16
NW = NC * NS
CHUNK = 64


def _chunk_size(tpw, width, n_bufs, esize=4, budget=400 * 1024):
    c = 128
    while c > 8 and (n_bufs * c * width * esize > budget or tpw % c != 0):
        c //= 2
    return c


def _make_dispatch_kernel(n, din, p_total):
    tpw = n // NW
    chunk = _chunk_size(tpw, din, 2)
    n_chunks = tpw // chunk
    mesh = plsc.VectorSubcoreMesh(core_axis_name="c", subcore_axis_name="s",
                                  num_cores=NC, num_subcores=NS)

    @functools.partial(
        pl.kernel,
        out_type=jax.ShapeDtypeStruct((p_total, din), jnp.int32),
        mesh=mesh,
        scratch_types=[
            pltpu.VMEM((chunk, din), jnp.int32),
            pltpu.VMEM((chunk, din), jnp.int32),
            pltpu.VMEM((n_chunks, chunk), jnp.int32),
            pltpu.VMEM((n_chunks, chunk), jnp.int32),
            pltpu.SemaphoreType.DMA,
            pltpu.SemaphoreType.DMA,
        ],
    )
    def dispatch(x_hbm, dpos_e_hbm, dpos_o_hbm, xs_hbm, rows0_v, rows1_v,
                 idxe_v, idxo_v, sem0, sem1):
        wid = lax.axis_index("s") * NC + lax.axis_index("c")
        pltpu.sync_copy(dpos_e_hbm.at[wid], idxe_v)
        pltpu.sync_copy(dpos_o_hbm.at[wid], idxo_v)
        rows = (rows0_v, rows1_v)
        sems = (sem0, sem1)
        pending = [None, None]
        for c in range(n_chunks):
            p = c % 2
            if pending[p] is not None:
                pending[p][0].wait()
                pending[p][1].wait()
            base = wid * tpw + c * chunk
            pltpu.sync_copy(x_hbm.at[pl.ds(base, chunk)], rows[p])
            c1 = pltpu.async_copy(rows[p], xs_hbm.at[idxe_v.at[c]], sems[p])
            c2 = pltpu.async_copy(rows[p], xs_hbm.at[idxo_v.at[c]], sems[p])
            pending[p] = (c1, c2)
        for p in range(2):
            if pending[p] is not None:
                pending[p][0].wait()
                pending[p][1].wait()

    return dispatch


def _make_combine_kernel(n, dout, p_total):
    tpw = n // NW
    chunk = _chunk_size(tpw, dout, 4)
    n_chunks = tpw // chunk
    mesh = plsc.VectorSubcoreMesh(core_axis_name="c", subcore_axis_name="s",
                                  num_cores=NC, num_subcores=NS)

    @functools.partial(
        pl.kernel,
        out_type=(jax.ShapeDtypeStruct((n, dout), jnp.float32),
                  jax.ShapeDtypeStruct((n, dout), jnp.float32)),
        mesh=mesh,
        scratch_types=[
            pltpu.VMEM((chunk, dout), jnp.float32),
            pltpu.VMEM((chunk, dout), jnp.float32),
            pltpu.VMEM((chunk, dout), jnp.float32),
            pltpu.VMEM((chunk, dout), jnp.float32),
            pltpu.VMEM((n_chunks, chunk), jnp.int32),
            pltpu.VMEM((n_chunks, chunk), jnp.int32),
            pltpu.SemaphoreType.DMA,
            pltpu.SemaphoreType.DMA,
        ],
    )
    def combine(yw_hbm, dpos_e_hbm, dpos_o_hbm, g0_hbm, g1_hbm, r0a_v, r1a_v,
                r0b_v, r1b_v, idxe_v, idxo_v, sem0, sem1):
        wid = lax.axis_index("s") * NC + lax.axis_index("c")
        pltpu.sync_copy(dpos_e_hbm.at[wid], idxe_v)
        pltpu.sync_copy(dpos_o_hbm.at[wid], idxo_v)
        bufs = ((r0a_v, r1a_v), (r0b_v, r1b_v))
        sems = (sem0, sem1)

        def issue(c):
            p = c % 2
            c1 = pltpu.async_copy(yw_hbm.at[idxe_v.at[c]], bufs[p][0], sems[p])
            c2 = pltpu.async_copy(yw_hbm.at[idxo_v.at[c]], bufs[p][1], sems[p])
            return (c1, c2)

        pending = issue(0)
        for c in range(n_chunks):
            p = c % 2
            nxt = issue(c + 1) if c + 1 < n_chunks else None
            pending[0].wait()
            pending[1].wait()
            base = wid * tpw + c * chunk
            pltpu.sync_copy(bufs[p][0], g0_hbm.at[pl.ds(base, chunk)])
            pltpu.sync_copy(bufs[p][1], g1_hbm.at[pl.ds(base, chunk)])
            pending = nxt

    return combine


def _rn_bf16_bits(xi):
    rounded = xi + 0x7FFF + jnp.bitwise_and(lax.shift_right_logical(xi, 16), 1)
    return jnp.bitwise_and(rounded, jnp.int32(-65536))


def _pack_kernel(x_ref, out_ref):
    blk, din = x_ref.shape
    dpk = out_ref.shape[1]
    x = x_ref[...]
    a = x[:, :dpk]
    b = x[:, dpk:]
    if din - dpk < dpk:
        b = jnp.concatenate(
            [b, jnp.zeros((blk, 2 * dpk - din), jnp.float32)], axis=1)
    ai = lax.bitcast_convert_type(a, jnp.int32)
    bi = lax.bitcast_convert_type(b, jnp.int32)
    out_ref[...] = jnp.bitwise_or(
        _rn_bf16_bits(ai), lax.shift_right_logical(_rn_bf16_bits(bi), 16))


def _grouped_matmul_kernel(be_ref, xs_ref, w_ref, b_ref, out_ref):
    w = xs_ref[...]
    dpk = w.shape[1]
    a = lax.bitcast_convert_type(jnp.bitwise_and(w, jnp.int32(-65536)),
                                 jnp.float32)
    bb = lax.bitcast_convert_type(lax.shift_left(w, 16), jnp.float32)
    y = (jnp.dot(a, w_ref[0, :dpk], preferred_element_type=jnp.float32) +
         jnp.dot(bb, w_ref[0, dpk:], preferred_element_type=jnp.float32))
    y = y + b_ref[0]
    out_ref[...] = y * jax.nn.sigmoid(y)


def _final_kernel(x_ref, wsh_ref, bsh_ref, g0_ref, g1_ref, w0_ref, w1_ref,
                  out_ref):
    d = out_ref.shape[1]
    y = jnp.dot(x_ref[...], wsh_ref[...], preferred_element_type=jnp.float32)
    y = y + bsh_ref[0]
    y = y * jax.nn.sigmoid(y)
    out_ref[...] = (y + w0_ref[...] * g0_ref[:, :d]
                    + w1_ref[...] * g1_ref[:, :d])


def _dispatch_plan(topk_idx, blk):
    n = topk_idx.shape[0]
    m = n * TOPK
    seg = 256
    p_total = m + N_EXP * blk
    flat_idx = topk_idx.reshape(m).astype(jnp.int32)

    oh = (flat_idx[:, None] == jnp.arange(N_EXP, dtype=jnp.int32)[None, :])
    oh = oh.astype(jnp.float32)
    oh3 = oh.reshape(m // seg, seg, N_EXP)
    tri = jnp.tril(jnp.ones((seg, seg), jnp.float32))
    within = jnp.einsum('ts,bso->bto', tri, oh3,
                        preferred_element_type=jnp.float32)
    bsum = oh3.sum(axis=1)
    excl = jnp.cumsum(bsum, axis=0) - bsum
    incl = (within + excl[:, None, :]).reshape(m, N_EXP)
    rank = (incl * oh).sum(axis=1) - 1.0

    counts = bsum.sum(axis=0)
    padded_counts = jnp.ceil(counts / blk) * blk
    pstart = jnp.cumsum(padded_counts) - padded_counts
    dpos = ((pstart[None, :] * oh).sum(axis=1) + rank).astype(jnp.int32)

    bounds = jnp.cumsum(padded_counts).astype(jnp.int32)
    blk_starts = lax.iota(jnp.int32, p_total // blk) * blk
    block_expert = jnp.minimum(
        jnp.searchsorted(bounds, blk_starts, side='right').astype(jnp.int32),
        N_EXP - 1)
    return dpos, block_expert


def _moe_stream(x, topk_w, topk_idx, W, b, Wsh, bsh, blk):
    n, din = x.shape
    dout0 = W.shape[-1]
    dout = ((dout0 + 127) // 128) * 128
    if dout != dout0:
        pad = dout - dout0
        W = jnp.pad(W, ((0, 0), (0, 0), (0, pad)))
        b = jnp.pad(b, ((0, 0), (0, pad)))
    dpos, block_expert = _dispatch_plan(topk_idx, blk)
    p_total = n * TOPK + N_EXP * blk
    dpos_e = dpos[0::2]
    dpos_o = dpos[1::2]
    tpw = n // NW

    din_pad = ((din + 255) // 256) * 256
    dpk = din_pad // 2
    pblk = 512
    xp = pl.pallas_call(
        _pack_kernel,
        grid=(n // pblk,),
        in_specs=[pl.BlockSpec((pblk, din), lambda i: (i, 0))],
        out_specs=pl.BlockSpec((pblk, dpk), lambda i: (i, 0)),
        out_shape=jax.ShapeDtypeStruct((n, dpk), jnp.int32),
    )(x)
    W_r = jnp.pad(W, ((0, 0), (0, din_pad - din), (0, 0))) \
        if din_pad != din else W

    cd = _chunk_size(tpw, dpk, 2)
    xs = _make_dispatch_kernel(n, dpk, p_total)(
        xp, dpos_e.reshape(NW, tpw // cd, cd), dpos_o.reshape(NW, tpw // cd, cd))

    grid_spec = pltpu.PrefetchScalarGridSpec(
        num_scalar_prefetch=1,
        grid=(p_total // blk,),
        in_specs=[
            pl.BlockSpec((blk, dpk), lambda i, be: (i, 0)),
            pl.BlockSpec((1, din_pad, dout), lambda i, be: (be[i], 0, 0)),
            pl.BlockSpec((1, 1, dout), lambda i, be: (be[i], 0, 0)),
        ],
        out_specs=pl.BlockSpec((blk, dout), lambda i, be: (i, 0)),
    )
    yw = pl.pallas_call(
        _grouped_matmul_kernel,
        grid_spec=grid_spec,
        out_shape=jax.ShapeDtypeStruct((p_total, dout), jnp.float32),
    )(block_expert, xs, W_r, b[:, None, :])

    cc = _chunk_size(tpw, dout, 4)
    g0, g1 = _make_combine_kernel(n, dout, p_total)(
        yw, dpos_e.reshape(NW, tpw // cc, cc), dpos_o.reshape(NW, tpw // cc, cc))

    blk2 = 512
    return pl.pallas_call(
        _final_kernel,
        grid=(n // blk2,),
        in_specs=[
            pl.BlockSpec((blk2, din), lambda i: (i, 0)),
            pl.BlockSpec((din, dout0), lambda i: (0, 0)),
            pl.BlockSpec((1, dout0), lambda i: (0, 0)),
            pl.BlockSpec((blk2, dout), lambda i: (i, 0)),
            pl.BlockSpec((blk2, dout), lambda i: (i, 0)),
            pl.BlockSpec((blk2, 1), lambda i: (i, 0)),
            pl.BlockSpec((blk2, 1), lambda i: (i, 0)),
        ],
        out_specs=pl.BlockSpec((blk2, dout0), lambda i: (i, 0)),
        out_shape=jax.ShapeDtypeStruct((n, dout0), jnp.float32),
    )(x, Wsh, bsh[None, :], g0, g1, topk_w[:, 0:1], topk_w[:, 1:2])


@jax.jit
def kernel(node_m1_input, node_m2_input, edge_input, angle_input,
           node_router_weights, node_router_indices,
           edge_router_weights, edge_router_indices,
           angle_router_weights, angle_router_indices,
           n2e_index, n2a_index,
           node_self_W, node_self_b, node_self_Wsh, node_self_bsh,
           node_sym_W, node_sym_b, node_sym_Wsh, node_sym_bsh,
           edge_W, edge_b, edge_Wsh, edge_bsh,
           angle_W, angle_b, angle_Wsh, angle_bsh):
    edge_idx = edge_router_indices[n2e_index]
    angle_idx = angle_router_indices[n2a_index]
    edge_w = edge_router_weights[n2e_index]
    angle_w = angle_router_weights[n2a_index]

    node_self_out = _moe_stream(node_m1_input, node_router_weights,
                                node_router_indices, node_self_W, node_self_b,
                                node_self_Wsh, node_self_bsh, 256)
    node_sym_out = _moe_stream(node_m2_input, node_router_weights,
                               node_router_indices, node_sym_W, node_sym_b,
                               node_sym_Wsh, node_sym_bsh, 256)
    edge_out = _moe_stream(edge_input, edge_w, edge_idx, edge_W, edge_b,
                           edge_Wsh, edge_bsh, 256)
    angle_out = _moe_stream(angle_input, angle_w, angle_idx, angle_W, angle_b,
                            angle_Wsh, angle_bsh, 256)
    return node_self_out, node_sym_out, edge_out, angle_out


# speedup vs baseline: 1.0895x; 1.0895x over previous
import functools

import jax
import jax.numpy as jnp
from jax import lax
from jax.experimental import pallas as pl
from jax.experimental.pallas import tpu as pltpu
from jax.experimental.pallas import tpu_sc as plsc

N_EXP = 8
TOPK = 2
NC = 2
NS = 16
NW = NC * NS
CHUNK = 64


def _make_dispatch_kernel(n, din, p_total):
    tpw = n // NW
    n_chunks = tpw // CHUNK
    mesh = plsc.VectorSubcoreMesh(core_axis_name="c", subcore_axis_name="s",
                                  num_cores=NC, num_subcores=NS)

    @functools.partial(
        pl.kernel,
        out_type=jax.ShapeDtypeStruct((p_total, din), jnp.float32),
        mesh=mesh,
        scratch_types=[
            pltpu.VMEM((CHUNK, din), jnp.float32),
            pltpu.VMEM((CHUNK,), jnp.int32),
            pltpu.VMEM((CHUNK,), jnp.int32),
            pltpu.SemaphoreType.DMA,
        ],
    )
    def dispatch(x_hbm, dpos_e_hbm, dpos_o_hbm, xs_hbm, rows_v, idxe_v,
                 idxo_v, sem):
        wid = lax.axis_index("s") * NC + lax.axis_index("c")

        def body(c, _):
            base = wid * tpw + c * CHUNK
            pltpu.sync_copy(x_hbm.at[pl.ds(base, CHUNK)], rows_v)
            pltpu.sync_copy(dpos_e_hbm.at[pl.ds(base, CHUNK)], idxe_v)
            pltpu.sync_copy(dpos_o_hbm.at[pl.ds(base, CHUNK)], idxo_v)
            c1 = pltpu.async_copy(rows_v, xs_hbm.at[idxe_v], sem)
            c2 = pltpu.async_copy(rows_v, xs_hbm.at[idxo_v], sem)
            c1.wait()
            c2.wait()
            return ()

        lax.fori_loop(0, n_chunks, body, ())

    return dispatch


def _make_combine_kernel(n, dout, p_total):
    tpw = n // NW
    n_chunks = tpw // CHUNK
    mesh = plsc.VectorSubcoreMesh(core_axis_name="c", subcore_axis_name="s",
                                  num_cores=NC, num_subcores=NS)

    @functools.partial(
        pl.kernel,
        out_type=(jax.ShapeDtypeStruct((n, dout), jnp.float32),
                  jax.ShapeDtypeStruct((n, dout), jnp.float32)),
        mesh=mesh,
        scratch_types=[
            pltpu.VMEM((CHUNK, dout), jnp.float32),
            pltpu.VMEM((CHUNK, dout), jnp.float32),
            pltpu.VMEM((CHUNK,), jnp.int32),
            pltpu.VMEM((CHUNK,), jnp.int32),
            pltpu.SemaphoreType.DMA,
        ],
    )
    def combine(yw_hbm, dpos_e_hbm, dpos_o_hbm, g0_hbm, g1_hbm, rows0_v,
                rows1_v, idxe_v, idxo_v, sem):
        wid = lax.axis_index("s") * NC + lax.axis_index("c")

        def body(c, _):
            base = wid * tpw + c * CHUNK
            pltpu.sync_copy(dpos_e_hbm.at[pl.ds(base, CHUNK)], idxe_v)
            pltpu.sync_copy(dpos_o_hbm.at[pl.ds(base, CHUNK)], idxo_v)
            c1 = pltpu.async_copy(yw_hbm.at[idxe_v], rows0_v, sem)
            c2 = pltpu.async_copy(yw_hbm.at[idxo_v], rows1_v, sem)
            c1.wait()
            c2.wait()
            pltpu.sync_copy(rows0_v, g0_hbm.at[pl.ds(base, CHUNK)])
            pltpu.sync_copy(rows1_v, g1_hbm.at[pl.ds(base, CHUNK)])
            return ()

        lax.fori_loop(0, n_chunks, body, ())

    return combine


def _grouped_matmul_kernel(be_ref, xs_ref, w_ref, b_ref, out_ref):
    y = jnp.dot(xs_ref[...], w_ref[0], preferred_element_type=jnp.float32)
    y = y + b_ref[0]
    out_ref[...] = y * jax.nn.sigmoid(y)


def _final_kernel(x_ref, wsh_ref, bsh_ref, g0_ref, g1_ref, w0_ref, w1_ref,
                  out_ref):
    d = out_ref.shape[1]
    y = jnp.dot(x_ref[...], wsh_ref[...], preferred_element_type=jnp.float32)
    y = y + bsh_ref[0]
    y = y * jax.nn.sigmoid(y)
    out_ref[...] = (y + w0_ref[...] * g0_ref[:, :d]
                    + w1_ref[...] * g1_ref[:, :d])


def _dispatch_plan(topk_idx, blk):
    n = topk_idx.shape[0]
    m = n * TOPK
    seg = 256
    p_total = m + N_EXP * blk
    flat_idx = topk_idx.reshape(m).astype(jnp.int32)

    oh = (flat_idx[:, None] == jnp.arange(N_EXP, dtype=jnp.int32)[None, :])
    oh = oh.astype(jnp.float32)
    oh3 = oh.reshape(m // seg, seg, N_EXP)
    tri = jnp.tril(jnp.ones((seg, seg), jnp.float32))
    within = jnp.einsum('ts,bso->bto', tri, oh3,
                        preferred_element_type=jnp.float32)
    bsum = oh3.sum(axis=1)
    excl = jnp.cumsum(bsum, axis=0) - bsum
    incl = (within + excl[:, None, :]).reshape(m, N_EXP)
    rank = (incl * oh).sum(axis=1) - 1.0

    counts = bsum.sum(axis=0)
    padded_counts = jnp.ceil(counts / blk) * blk
    pstart = jnp.cumsum(padded_counts) - padded_counts
    dpos = ((pstart[None, :] * oh).sum(axis=1) + rank).astype(jnp.int32)

    bounds = jnp.cumsum(padded_counts).astype(jnp.int32)
    blk_starts = lax.iota(jnp.int32, p_total // blk) * blk
    block_expert = jnp.minimum(
        jnp.searchsorted(bounds, blk_starts, side='right').astype(jnp.int32),
        N_EXP - 1)
    return dpos, block_expert


def _moe_stream(x, topk_w, topk_idx, W, b, Wsh, bsh, blk):
    n, din = x.shape
    dout0 = W.shape[-1]
    dout = ((dout0 + 127) // 128) * 128
    if dout != dout0:
        pad = dout - dout0
        W = jnp.pad(W, ((0, 0), (0, 0), (0, pad)))
        b = jnp.pad(b, ((0, 0), (0, pad)))
    dpos, block_expert = _dispatch_plan(topk_idx, blk)
    p_total = n * TOPK + N_EXP * blk
    dpos_e = dpos[0::2]
    dpos_o = dpos[1::2]

    xs = _make_dispatch_kernel(n, din, p_total)(x, dpos_e, dpos_o)

    grid_spec = pltpu.PrefetchScalarGridSpec(
        num_scalar_prefetch=1,
        grid=(p_total // blk,),
        in_specs=[
            pl.BlockSpec((blk, din), lambda i, be: (i, 0)),
            pl.BlockSpec((1, din, dout), lambda i, be: (be[i], 0, 0)),
            pl.BlockSpec((1, 1, dout), lambda i, be: (be[i], 0, 0)),
        ],
        out_specs=pl.BlockSpec((blk, dout), lambda i, be: (i, 0)),
    )
    yw = pl.pallas_call(
        _grouped_matmul_kernel,
        grid_spec=grid_spec,
        out_shape=jax.ShapeDtypeStruct((p_total, dout), jnp.float32),
    )(block_expert, xs, W, b[:, None, :])

    g0, g1 = _make_combine_kernel(n, dout, p_total)(yw, dpos_e, dpos_o)

    blk2 = 512
    return pl.pallas_call(
        _final_kernel,
        grid=(n // blk2,),
        in_specs=[
            pl.BlockSpec((blk2, din), lambda i: (i, 0)),
            pl.BlockSpec((din, dout0), lambda i: (0, 0)),
            pl.BlockSpec((1, dout0), lambda i: (0, 0)),
            pl.BlockSpec((blk2, dout), lambda i: (i, 0)),
            pl.BlockSpec((blk2, dout), lambda i: (i, 0)),
            pl.BlockSpec((blk2, 1), lambda i: (i, 0)),
            pl.BlockSpec((blk2, 1), lambda i: (i, 0)),
        ],
        out_specs=pl.BlockSpec((blk2, dout0), lambda i: (i, 0)),
        out_shape=jax.ShapeDtypeStruct((n, dout0), jnp.float32),
    )(x, Wsh, bsh[None, :], g0, g1, topk_w[:, 0:1], topk_w[:, 1:2])


@jax.jit
def kernel(node_m1_input, node_m2_input, edge_input, angle_input,
           node_router_weights, node_router_indices,
           edge_router_weights, edge_router_indices,
           angle_router_weights, angle_router_indices,
           n2e_index, n2a_index,
           node_self_W, node_self_b, node_self_Wsh, node_self_bsh,
           node_sym_W, node_sym_b, node_sym_Wsh, node_sym_bsh,
           edge_W, edge_b, edge_Wsh, edge_bsh,
           angle_W, angle_b, angle_Wsh, angle_bsh):
    edge_idx = edge_router_indices[n2e_index]
    angle_idx = angle_router_indices[n2a_index]
    edge_w = edge_router_weights[n2e_index]
    angle_w = angle_router_weights[n2a_index]

    node_self_out = _moe_stream(node_m1_input, node_router_weights,
                                node_router_indices, node_self_W, node_self_b,
                                node_self_Wsh, node_self_bsh, 256)
    node_sym_out = _moe_stream(node_m2_input, node_router_weights,
                               node_router_indices, node_sym_W, node_sym_b,
                               node_sym_Wsh, node_sym_bsh, 256)
    edge_out = _moe_stream(edge_input, edge_w, edge_idx, edge_W, edge_b,
                           edge_Wsh, edge_bsh, 256)
    angle_out = _moe_stream(angle_input, angle_w, angle_idx, angle_W, angle_b,
                            angle_Wsh, angle_bsh, 256)
    return node_self_out, node_sym_out, edge_out, angle_out


# speedup vs baseline: 1.0910x; 1.0014x over previous
import functools

import jax
import jax.numpy as jnp
from jax import lax
from jax.experimental import pallas as pl
from jax.experimental.pallas import tpu as pltpu
from jax.experimental.pallas import tpu_sc as plsc

N_EXP = 8
TOPK = 2
NC = 2
NS = 16
NW = NC * NS
CHUNK = 64


def _make_dispatch_kernel(n, din, p_total):
    tpw = n // NW
    chunk = 128 if 128 * din * 4 <= 400 * 1024 and tpw % 128 == 0 else CHUNK
    n_chunks = tpw // chunk
    mesh = plsc.VectorSubcoreMesh(core_axis_name="c", subcore_axis_name="s",
                                  num_cores=NC, num_subcores=NS)

    @functools.partial(
        pl.kernel,
        out_type=jax.ShapeDtypeStruct((p_total, din), jnp.float32),
        mesh=mesh,
        scratch_types=[
            pltpu.VMEM((chunk, din), jnp.float32),
            pltpu.VMEM((chunk,), jnp.int32),
            pltpu.VMEM((chunk,), jnp.int32),
            pltpu.SemaphoreType.DMA,
        ],
    )
    def dispatch(x_hbm, dpos_e_hbm, dpos_o_hbm, xs_hbm, rows_v, idxe_v,
                 idxo_v, sem):
        wid = lax.axis_index("s") * NC + lax.axis_index("c")

        def body(c, _):
            base = wid * tpw + c * chunk
            pltpu.sync_copy(x_hbm.at[pl.ds(base, chunk)], rows_v)
            pltpu.sync_copy(dpos_e_hbm.at[pl.ds(base, chunk)], idxe_v)
            pltpu.sync_copy(dpos_o_hbm.at[pl.ds(base, chunk)], idxo_v)
            c1 = pltpu.async_copy(rows_v, xs_hbm.at[idxe_v], sem)
            c2 = pltpu.async_copy(rows_v, xs_hbm.at[idxo_v], sem)
            c1.wait()
            c2.wait()
            return ()

        lax.fori_loop(0, n_chunks, body, ())

    return dispatch


def _make_combine_kernel(n, dout, p_total):
    tpw = n // NW
    chunk = 128 if 2 * 128 * dout * 4 <= 400 * 1024 and tpw % 128 == 0 else CHUNK
    n_chunks = tpw // chunk
    mesh = plsc.VectorSubcoreMesh(core_axis_name="c", subcore_axis_name="s",
                                  num_cores=NC, num_subcores=NS)

    @functools.partial(
        pl.kernel,
        out_type=(jax.ShapeDtypeStruct((n, dout), jnp.float32),
                  jax.ShapeDtypeStruct((n, dout), jnp.float32)),
        mesh=mesh,
        scratch_types=[
            pltpu.VMEM((chunk, dout), jnp.float32),
            pltpu.VMEM((chunk, dout), jnp.float32),
            pltpu.VMEM((chunk,), jnp.int32),
            pltpu.VMEM((chunk,), jnp.int32),
            pltpu.SemaphoreType.DMA,
        ],
    )
    def combine(yw_hbm, dpos_e_hbm, dpos_o_hbm, g0_hbm, g1_hbm, rows0_v,
                rows1_v, idxe_v, idxo_v, sem):
        wid = lax.axis_index("s") * NC + lax.axis_index("c")

        def body(c, _):
            base = wid * tpw + c * chunk
            pltpu.sync_copy(dpos_e_hbm.at[pl.ds(base, chunk)], idxe_v)
            pltpu.sync_copy(dpos_o_hbm.at[pl.ds(base, chunk)], idxo_v)
            c1 = pltpu.async_copy(yw_hbm.at[idxe_v], rows0_v, sem)
            c2 = pltpu.async_copy(yw_hbm.at[idxo_v], rows1_v, sem)
            c1.wait()
            c2.wait()
            pltpu.sync_copy(rows0_v, g0_hbm.at[pl.ds(base, chunk)])
            pltpu.sync_copy(rows1_v, g1_hbm.at[pl.ds(base, chunk)])
            return ()

        lax.fori_loop(0, n_chunks, body, ())

    return combine


def _grouped_matmul_kernel(be_ref, xs_ref, w_ref, b_ref, out_ref):
    y = jnp.dot(xs_ref[...], w_ref[0], preferred_element_type=jnp.float32)
    y = y + b_ref[0]
    out_ref[...] = y * jax.nn.sigmoid(y)


def _final_kernel(x_ref, wsh_ref, bsh_ref, g0_ref, g1_ref, w0_ref, w1_ref,
                  out_ref):
    d = out_ref.shape[1]
    y = jnp.dot(x_ref[...], wsh_ref[...], preferred_element_type=jnp.float32)
    y = y + bsh_ref[0]
    y = y * jax.nn.sigmoid(y)
    out_ref[...] = (y + w0_ref[...] * g0_ref[:, :d]
                    + w1_ref[...] * g1_ref[:, :d])


def _dispatch_plan(topk_idx, blk):
    n = topk_idx.shape[0]
    m = n * TOPK
    seg = 256
    p_total = m + N_EXP * blk
    flat_idx = topk_idx.reshape(m).astype(jnp.int32)

    oh = (flat_idx[:, None] == jnp.arange(N_EXP, dtype=jnp.int32)[None, :])
    oh = oh.astype(jnp.float32)
    oh3 = oh.reshape(m // seg, seg, N_EXP)
    tri = jnp.tril(jnp.ones((seg, seg), jnp.float32))
    within = jnp.einsum('ts,bso->bto', tri, oh3,
                        preferred_element_type=jnp.float32)
    bsum = oh3.sum(axis=1)
    excl = jnp.cumsum(bsum, axis=0) - bsum
    incl = (within + excl[:, None, :]).reshape(m, N_EXP)
    rank = (incl * oh).sum(axis=1) - 1.0

    counts = bsum.sum(axis=0)
    padded_counts = jnp.ceil(counts / blk) * blk
    pstart = jnp.cumsum(padded_counts) - padded_counts
    dpos = ((pstart[None, :] * oh).sum(axis=1) + rank).astype(jnp.int32)

    bounds = jnp.cumsum(padded_counts).astype(jnp.int32)
    blk_starts = lax.iota(jnp.int32, p_total // blk) * blk
    block_expert = jnp.minimum(
        jnp.searchsorted(bounds, blk_starts, side='right').astype(jnp.int32),
        N_EXP - 1)
    return dpos, block_expert


def _moe_stream(x, topk_w, topk_idx, W, b, Wsh, bsh, blk):
    n, din = x.shape
    dout0 = W.shape[-1]
    dout = ((dout0 + 127) // 128) * 128
    if dout != dout0:
        pad = dout - dout0
        W = jnp.pad(W, ((0, 0), (0, 0), (0, pad)))
        b = jnp.pad(b, ((0, 0), (0, pad)))
    dpos, block_expert = _dispatch_plan(topk_idx, blk)
    p_total = n * TOPK + N_EXP * blk
    dpos_e = dpos[0::2]
    dpos_o = dpos[1::2]

    xs = _make_dispatch_kernel(n, din, p_total)(x, dpos_e, dpos_o)

    grid_spec = pltpu.PrefetchScalarGridSpec(
        num_scalar_prefetch=1,
        grid=(p_total // blk,),
        in_specs=[
            pl.BlockSpec((blk, din), lambda i, be: (i, 0)),
            pl.BlockSpec((1, din, dout), lambda i, be: (be[i], 0, 0)),
            pl.BlockSpec((1, 1, dout), lambda i, be: (be[i], 0, 0)),
        ],
        out_specs=pl.BlockSpec((blk, dout), lambda i, be: (i, 0)),
    )
    yw = pl.pallas_call(
        _grouped_matmul_kernel,
        grid_spec=grid_spec,
        out_shape=jax.ShapeDtypeStruct((p_total, dout), jnp.float32),
    )(block_expert, xs, W, b[:, None, :])

    g0, g1 = _make_combine_kernel(n, dout, p_total)(yw, dpos_e, dpos_o)

    blk2 = 512
    return pl.pallas_call(
        _final_kernel,
        grid=(n // blk2,),
        in_specs=[
            pl.BlockSpec((blk2, din), lambda i: (i, 0)),
            pl.BlockSpec((din, dout0), lambda i: (0, 0)),
            pl.BlockSpec((1, dout0), lambda i: (0, 0)),
            pl.BlockSpec((blk2, dout), lambda i: (i, 0)),
            pl.BlockSpec((blk2, dout), lambda i: (i, 0)),
            pl.BlockSpec((blk2, 1), lambda i: (i, 0)),
            pl.BlockSpec((blk2, 1), lambda i: (i, 0)),
        ],
        out_specs=pl.BlockSpec((blk2, dout0), lambda i: (i, 0)),
        out_shape=jax.ShapeDtypeStruct((n, dout0), jnp.float32),
    )(x, Wsh, bsh[None, :], g0, g1, topk_w[:, 0:1], topk_w[:, 1:2])


@jax.jit
def kernel(node_m1_input, node_m2_input, edge_input, angle_input,
           node_router_weights, node_router_indices,
           edge_router_weights, edge_router_indices,
           angle_router_weights, angle_router_indices,
           n2e_index, n2a_index,
           node_self_W, node_self_b, node_self_Wsh, node_self_bsh,
           node_sym_W, node_sym_b, node_sym_Wsh, node_sym_bsh,
           edge_W, edge_b, edge_Wsh, edge_bsh,
           angle_W, angle_b, angle_Wsh, angle_bsh):
    edge_idx = edge_router_indices[n2e_index]
    angle_idx = angle_router_indices[n2a_index]
    edge_w = edge_router_weights[n2e_index]
    angle_w = angle_router_weights[n2a_index]

    node_self_out = _moe_stream(node_m1_input, node_router_weights,
                                node_router_indices, node_self_W, node_self_b,
                                node_self_Wsh, node_self_bsh, 256)
    node_sym_out = _moe_stream(node_m2_input, node_router_weights,
                               node_router_indices, node_sym_W, node_sym_b,
                               node_sym_Wsh, node_sym_bsh, 256)
    edge_out = _moe_stream(edge_input, edge_w, edge_idx, edge_W, edge_b,
                           edge_Wsh, edge_bsh, 256)
    angle_out = _moe_stream(angle_input, angle_w, angle_idx, angle_W, angle_b,
                            angle_Wsh, angle_bsh, 256)
    return node_self_out, node_sym_out, edge_out, angle_out
